# Initial kernel scaffold; baseline (speedup 1.0000x reference)
import jax
import jax.numpy as jnp
from jax.experimental import pallas as pl


def kernel(x, positions, pe):
    raise NotImplementedError("write your pallas kernel here")


# speedup vs baseline: 2.9483x; 2.9483x over previous
import functools

import jax
import jax.numpy as jnp
from jax import lax
from jax.experimental import pallas as pl
from jax.experimental.pallas import tpu as pltpu
from jax.experimental.pallas import tpu_sc as plsc

HALF = 64
LANES = 16
N_TILES = 32
W = 128


def _lookup_add(x2, idx, pe):
    R = x2.shape[0]
    rows_per_tile = R // N_TILES
    n_windows = rows_per_tile // W

    mesh = plsc.VectorSubcoreMesh(core_axis_name="c", subcore_axis_name="s")

    @functools.partial(
        pl.kernel,
        out_type=jax.ShapeDtypeStruct((R, HALF), jnp.float32),
        mesh=mesh,
        compiler_params=pltpu.CompilerParams(use_tc_tiling_on_sc=False),
        scratch_types=[
            pltpu.VMEM((W,), jnp.int32),
            pltpu.VMEM((W, HALF), jnp.float32),
            pltpu.VMEM((W, HALF), jnp.float32),
            pltpu.SemaphoreType.DMA,
        ],
    )
    def k(x_hbm, idx_hbm, pe_hbm, out_hbm, idx_v, rows_v, x_v, sem):
        wid = lax.axis_index("s") * 2 + lax.axis_index("c")
        tile_base = wid * rows_per_tile

        @pl.loop(0, n_windows)
        def _(w):
            base = tile_base + w * W
            pltpu.sync_copy(idx_hbm.at[pl.ds(base, W)], idx_v)
            gather = pltpu.async_copy(pe_hbm.at[idx_v], rows_v, sem)
            pltpu.sync_copy(x_hbm.at[pl.ds(base, W)], x_v)
            gather.wait()

            @pl.loop(0, W)
            def _(r):
                for j in range(HALF // LANES):
                    s = pl.ds(j * LANES, LANES)
                    x_v[r, s] = x_v[r, s] + rows_v[r, s]

            pltpu.sync_copy(x_v, out_hbm.at[pl.ds(base, W)])

    return k(x2, idx, pe)


def kernel(x, positions, pe):
    B, T, D = x.shape
    R = B * T * 2
    x2 = x.reshape(R, HALF)
    idx = positions.reshape(R)
    out2 = _lookup_add(x2, idx, pe)
    return out2.reshape(B, T, D)


# speedup vs baseline: 3.7744x; 1.2802x over previous
import functools

import jax
import jax.numpy as jnp
from jax import lax
from jax.experimental import pallas as pl
from jax.experimental.pallas import tpu as pltpu
from jax.experimental.pallas import tpu_sc as plsc

HALF = 64
LANES = 16
N_TILES = 32
W = 128


def _lookup_add(x2, idx, pe):
    R = x2.shape[0]
    rows_per_tile = R // N_TILES
    n_windows = rows_per_tile // W

    mesh = plsc.VectorSubcoreMesh(core_axis_name="c", subcore_axis_name="s")

    @functools.partial(
        pl.kernel,
        out_type=jax.ShapeDtypeStruct((R, HALF), jnp.float32),
        mesh=mesh,
        compiler_params=pltpu.CompilerParams(use_tc_tiling_on_sc=False),
        scratch_types=[
            pltpu.VMEM((W,), jnp.int32),
            pltpu.VMEM((W,), jnp.int32),
            pltpu.VMEM((W, HALF), jnp.float32),
            pltpu.VMEM((W, HALF), jnp.float32),
            pltpu.VMEM((W, HALF), jnp.float32),
            pltpu.VMEM((W, HALF), jnp.float32),
            pltpu.SemaphoreType.DMA((2,)),
            pltpu.SemaphoreType.DMA((2,)),
            pltpu.SemaphoreType.DMA((2,)),
            pltpu.SemaphoreType.DMA((2,)),
        ],
    )
    def k(x_hbm, idx_hbm, pe_hbm, out_hbm,
          idx0, idx1, rows0, rows1, xv0, xv1, isem, gsem, xsem, osem):
        wid = lax.axis_index("s") * 2 + lax.axis_index("c")
        tile_base = wid * rows_per_tile
        idx_b = (idx0, idx1)
        rows_b = (rows0, rows1)
        x_b = (xv0, xv1)

        def idx_copy(w, b):
            return pltpu.make_async_copy(
                idx_hbm.at[pl.ds(tile_base + w * W, W)], idx_b[b], isem.at[b])

        def gather_copy(w, b):
            del w
            return pltpu.make_async_copy(pe_hbm.at[idx_b[b]], rows_b[b], gsem.at[b])

        def x_copy(w, b):
            return pltpu.make_async_copy(
                x_hbm.at[pl.ds(tile_base + w * W, W)], x_b[b], xsem.at[b])

        def out_copy(w, b):
            return pltpu.make_async_copy(
                x_b[b], out_hbm.at[pl.ds(tile_base + w * W, W)], osem.at[b])

        idx_copy(0, 0).start()
        idx_copy(1, 1).start()
        idx_copy(0, 0).wait()
        gather_copy(0, 0).start()
        x_copy(0, 0).start()

        @pl.loop(0, n_windows // 2)
        def _(h):
            for b in (0, 1):
                w = 2 * h + b
                nb = 1 - b

                @pl.when(w >= 1)
                def _():
                    out_copy(w - 1, nb).wait()

                @pl.when(w + 1 < n_windows)
                def _():
                    idx_copy(w + 1, nb).wait()
                    gather_copy(w + 1, nb).start()
                    x_copy(w + 1, nb).start()

                gather_copy(w, b).wait()
                x_copy(w, b).wait()

                @pl.when(w + 2 < n_windows)
                def _():
                    idx_copy(w + 2, b).start()

                @pl.loop(0, W, unroll=8)
                def _(r):
                    for j in range(HALF // LANES):
                        s = pl.ds(j * LANES, LANES)
                        plsc.addupdate(x_b[b].at[r, s], rows_b[b][r, s])

                out_copy(w, b).start()

        out_copy(n_windows - 1, 1).wait()

    return k(x2, idx, pe)


def kernel(x, positions, pe):
    B, T, D = x.shape
    R = B * T * 2
    x2 = x.reshape(R, HALF)
    idx = positions.reshape(R)
    out2 = _lookup_add(x2, idx, pe)
    return out2.reshape(B, T, D)
